# baseline (speedup 1.0000x reference)
import functools

import jax
import jax.numpy as jnp
from jax import lax
from jax.experimental import pallas as pl
from jax.experimental.pallas import tpu as pltpu
from jax.experimental.pallas import tpu_sc as plsc

H = 56
W = 56
C = 256
NROI = 512
P = 7
NBIN = P * P
NTAB = 16
HW = H * W

NC = 2
NS = 16
NWORK = NC * NS
RPW = NROI // NWORK
ROISEG = 224
SEGOFF = (0, 56, 112, 168)
SEGLEN = (56, 56, 56, 56)
RROWS = 224
CHALF = C // 2


def _tables_body(fmt_ref, out_ref):
    kh = pl.program_id(0)
    X = fmt_ref[...]
    for k in range(3):
        s = 1 << k
        sh = jnp.concatenate(
            [X[s:], jnp.broadcast_to(X[-1:], (s, W, C))], axis=0)
        X = jnp.where(kh > k, jnp.maximum(X, sh), X)
    Y = X
    for kw in range(4):
        if kw > 0:
            s = 1 << (kw - 1)
            sh = jnp.concatenate(
                [Y[:, s:], jnp.broadcast_to(Y[:, -1:], (H, s, C))], axis=1)
            Y = jnp.maximum(Y, sh)
        out_ref[0, kw] = Y


def _build_tables(FMt):
    return pl.pallas_call(
        _tables_body,
        grid=(4,),
        in_specs=[pl.BlockSpec((H, W, C), lambda g: (0, 0, 0))],
        out_specs=pl.BlockSpec((1, 4, H, W, C), lambda g: (g, 0, 0, 0, 0)),
        out_shape=jax.ShapeDtypeStruct((4, 4, H, W, C), jnp.float32),
    )(FMt)


def _rint_nonneg(x):
    fl = x.astype(jnp.int32)
    fr = x - fl.astype(jnp.float32)
    odd = (fl & 1) == 1
    up = (fr > 0.5) | ((fr == 0.5) & odd)
    return fl + up.astype(jnp.int32)


def _sc_body(tabs, roist, out, rv, idxb, rows, outT, semA, semB):
    cid = lax.axis_index("c")
    sid = lax.axis_index("s")
    wid = sid * NC + cid
    base = wid * RPW

    for d in range(4):
        pltpu.sync_copy(roist.at[d, pl.ds(base * 1, RPW)], rv.at[d])

    lane = jnp.arange(RPW, dtype=jnp.int32)
    zero = jnp.zeros((RPW,), jnp.int32)

    for q in range(4):
        for k in range(NBIN, 56):
            plsc.store_scatter(idxb, [lane * ROISEG + (SEGOFF[q] + k)], zero)

    fi = rv[0]
    fj = rv[1]
    fh = rv[2]
    fw = rv[3]
    y0 = jnp.clip(_rint_nonneg(fi * float(H)), 0, H - 1)
    x0 = jnp.clip(_rint_nonneg(fj * float(W)), 0, W - 1)
    rh = jnp.minimum(jnp.maximum(_rint_nonneg(fh * float(H)), 1), H - y0)
    rw = jnp.minimum(jnp.maximum(_rint_nonneg(fw * float(W)), 1), W - x0)

    def edges(p, v0, rv_):
        s = v0 + (p * rv_) // P
        e = v0 + ((p + 1) * rv_ + (P - 1)) // P
        e = jnp.maximum(e, s + 1)
        d = e - s
        pw = jnp.where(d >= 8, 8, jnp.where(d >= 4, 4, jnp.where(d >= 2, 2, 1)))
        kk = (
            (d >= 2).astype(jnp.int32)
            + (d >= 4).astype(jnp.int32)
            + (d >= 8).astype(jnp.int32)
        )
        return s, e - pw, kk

    hA = []
    wA = []
    for p in range(P):
        hA.append(edges(p, y0, rh))
        wA.append(edges(p, x0, rw))

    for py in range(P):
        r0, r1, kh = hA[py]
        for px in range(P):
            c0, c1, kw = wA[px]
            tb = (kh * 4 + kw) * HW
            b = py * P + px
            for q, (rr, cc) in enumerate(((r0, c0), (r0, c1), (r1, c0), (r1, c1))):
                plsc.store_scatter(
                    idxb, [lane * ROISEG + (SEGOFF[q] + b)], tb + rr * W + cc)

    sems = (semA, semB)
    ivec = jnp.arange(16, dtype=jnp.int32) * NBIN

    def issue(r, buf):
        for g in range(4):
            pltpu.async_copy(
                tabs.at[idxb.at[pl.ds(r * ROISEG + SEGOFF[g], SEGLEN[g])]],
                rows.at[buf, pl.ds(SEGOFF[g], SEGLEN[g])], sems[buf])

    def drain(buf):
        for g in range(4):
            pltpu.make_async_copy(
                tabs.at[pl.ds(0, SEGLEN[g])],
                rows.at[buf, pl.ds(SEGOFF[g], SEGLEN[g])], sems[buf]).wait()

    def compute(r, buf):
        for half in range(2):
            def per_row(py, carry, half=half):
                b0 = py * P
                for px in range(P):
                    bvec = ivec + (b0 + px)
                    for vl in range(CHALF // 16):
                        v = half * (CHALF // 16) + vl
                        m = jnp.maximum(
                            jnp.maximum(
                                rows[buf, SEGOFF[0] + b0 + px, pl.ds(16 * v, 16)],
                                rows[buf, SEGOFF[1] + b0 + px, pl.ds(16 * v, 16)]),
                            jnp.maximum(
                                rows[buf, SEGOFF[2] + b0 + px, pl.ds(16 * v, 16)],
                                rows[buf, SEGOFF[3] + b0 + px, pl.ds(16 * v, 16)]))
                        plsc.store_scatter(outT, [bvec + (16 * NBIN * vl)], m)
                return carry

            lax.fori_loop(0, P, per_row, 0)
            pltpu.sync_copy(
                outT, out.at[base + r, pl.ds(half * (CHALF * NBIN), CHALF * NBIN)])

    issue(0, 0)
    issue(1, 1)

    def pair(g, carry):
        ra = 2 * g
        drain(0)
        compute(ra, 0)

        @pl.when(ra + 2 < RPW)
        def _():
            issue(ra + 2, 0)

        drain(1)
        compute(ra + 1, 1)

        @pl.when(ra + 3 < RPW)
        def _():
            issue(ra + 3, 1)

        return carry

    lax.fori_loop(0, RPW // 2, pair, 0)


def _sc_pool(tabs, roist):
    mesh = plsc.VectorSubcoreMesh(core_axis_name="c", subcore_axis_name="s")
    return pl.kernel(
        _sc_body,
        mesh=mesh,
        compiler_params=pltpu.CompilerParams(needs_layout_passes=False),
        out_type=jax.ShapeDtypeStruct((NROI, C * NBIN), jnp.float32),
        scratch_types=[
            pltpu.VMEM((4, RPW), jnp.float32),
            pltpu.VMEM((RPW * ROISEG,), jnp.int32),
            pltpu.VMEM((2, RROWS, C), jnp.float32),
            pltpu.VMEM((CHALF * NBIN,), jnp.float32),
            pltpu.SemaphoreType.DMA,
            pltpu.SemaphoreType.DMA,
        ],
    )(tabs, roist)


@jax.jit
def kernel(FM, rois):
    FMt = jnp.transpose(FM, (1, 2, 0))
    tabs = _build_tables(FMt).reshape(NTAB * HW, C)
    roist = jnp.transpose(rois, (1, 0))
    out = _sc_pool(tabs, roist)
    return out.reshape(NROI, C, P, P)


# speedup vs baseline: 1.0040x; 1.0040x over previous
import functools

import jax
import jax.numpy as jnp
from jax import lax
from jax.experimental import pallas as pl
from jax.experimental.pallas import tpu as pltpu
from jax.experimental.pallas import tpu_sc as plsc

H = 56
W = 56
C = 256
NROI = 512
P = 7
NBIN = P * P
NTAB = 16
HW = H * W

NC = 2
NS = 16
NWORK = NC * NS
RPW = NROI // NWORK
GROWS = 112
NSEG = 2
RROWS = 224
CHALF = C // 2


def _tables_body(fmt_ref, out_ref):
    kh = pl.program_id(0)
    X = fmt_ref[...]
    for k in range(3):
        s = 1 << k
        sh = jnp.concatenate(
            [X[s:], jnp.broadcast_to(X[-1:], (s, W, C))], axis=0)
        X = jnp.where(kh > k, jnp.maximum(X, sh), X)
    Y = X
    for kw in range(4):
        if kw > 0:
            s = 1 << (kw - 1)
            sh = jnp.concatenate(
                [Y[:, s:], jnp.broadcast_to(Y[:, -1:], (H, s, C))], axis=1)
            Y = jnp.maximum(Y, sh)
        out_ref[0, kw] = Y


def _build_tables(FMt):
    return pl.pallas_call(
        _tables_body,
        grid=(4,),
        in_specs=[pl.BlockSpec((H, W, C), lambda g: (0, 0, 0))],
        out_specs=pl.BlockSpec((1, 4, H, W, C), lambda g: (g, 0, 0, 0, 0)),
        out_shape=jax.ShapeDtypeStruct((4, 4, H, W, C), jnp.float32),
    )(FMt)


def _rint_nonneg(x):
    fl = x.astype(jnp.int32)
    fr = x - fl.astype(jnp.float32)
    odd = (fl & 1) == 1
    up = (fr > 0.5) | ((fr == 0.5) & odd)
    return fl + up.astype(jnp.int32)


def _sc_body(tabs, roist, out, rv, idxb, rows, outT, semA, semB):
    cid = lax.axis_index("c")
    sid = lax.axis_index("s")
    wid = sid * NC + cid
    base = wid * RPW

    for d in range(4):
        pltpu.sync_copy(roist.at[d, pl.ds(base * 1, RPW)], rv.at[d])

    lane = jnp.arange(RPW, dtype=jnp.int32)
    zero = jnp.zeros((RPW,), jnp.int32)

    def idx_store(slot, val):
        g, k = divmod(slot, GROWS)
        plsc.store_scatter(
            idxb,
            [lane, jnp.full((RPW,), g, jnp.int32),
             jnp.full((RPW,), k, jnp.int32)],
            val)

    for s in range(4 * NBIN, NSEG * GROWS):
        idx_store(s, zero)

    fi = rv[0]
    fj = rv[1]
    fh = rv[2]
    fw = rv[3]
    y0 = jnp.clip(_rint_nonneg(fi * float(H)), 0, H - 1)
    x0 = jnp.clip(_rint_nonneg(fj * float(W)), 0, W - 1)
    rh = jnp.minimum(jnp.maximum(_rint_nonneg(fh * float(H)), 1), H - y0)
    rw = jnp.minimum(jnp.maximum(_rint_nonneg(fw * float(W)), 1), W - x0)

    def edges(p, v0, rv_):
        s = v0 + (p * rv_) // P
        e = v0 + ((p + 1) * rv_ + (P - 1)) // P
        e = jnp.maximum(e, s + 1)
        d = e - s
        pw = jnp.where(d >= 8, 8, jnp.where(d >= 4, 4, jnp.where(d >= 2, 2, 1)))
        kk = (
            (d >= 2).astype(jnp.int32)
            + (d >= 4).astype(jnp.int32)
            + (d >= 8).astype(jnp.int32)
        )
        return s, e - pw, kk

    hA = []
    wA = []
    for p in range(P):
        hA.append(edges(p, y0, rh))
        wA.append(edges(p, x0, rw))

    for py in range(P):
        r0, r1, kh = hA[py]
        for px in range(P):
            c0, c1, kw = wA[px]
            tb = (kh * 4 + kw) * HW
            b = py * P + px
            for q, (rr, cc) in enumerate(((r0, c0), (r0, c1), (r1, c0), (r1, c1))):
                idx_store(q * NBIN + b, tb + rr * W + cc)

    sems = (semA, semB)
    ivec = jnp.arange(16, dtype=jnp.int32) * NBIN

    def issue(r, buf):
        for g in range(NSEG):
            pltpu.async_copy(
                tabs.at[idxb.at[r, g]],
                rows.at[buf, pl.ds(g * GROWS, GROWS)], sems[buf])

    def drain(buf):
        for g in range(NSEG):
            pltpu.make_async_copy(
                tabs.at[pl.ds(0, GROWS)],
                rows.at[buf, pl.ds(g * GROWS, GROWS)], sems[buf]).wait()

    def compute(r, buf):
        for half in range(2):
            def per_row(py, carry, half=half):
                b0 = py * P
                for px in range(P):
                    bvec = ivec + (b0 + px)
                    for vl in range(CHALF // 16):
                        v = half * (CHALF // 16) + vl
                        m = jnp.maximum(
                            jnp.maximum(
                                rows[buf, b0 + px, pl.ds(16 * v, 16)],
                                rows[buf, NBIN + b0 + px, pl.ds(16 * v, 16)]),
                            jnp.maximum(
                                rows[buf, 2 * NBIN + b0 + px, pl.ds(16 * v, 16)],
                                rows[buf, 3 * NBIN + b0 + px, pl.ds(16 * v, 16)]))
                        plsc.store_scatter(outT, [bvec + (16 * NBIN * vl)], m)
                return carry

            lax.fori_loop(0, P, per_row, 0)
            pltpu.sync_copy(
                outT, out.at[base + r, pl.ds(half * (CHALF * NBIN), CHALF * NBIN)])

    issue(0, 0)
    issue(1, 1)

    def pair(g, carry):
        ra = 2 * g
        drain(0)
        compute(ra, 0)

        @pl.when(ra + 2 < RPW)
        def _():
            issue(ra + 2, 0)

        drain(1)
        compute(ra + 1, 1)

        @pl.when(ra + 3 < RPW)
        def _():
            issue(ra + 3, 1)

        return carry

    lax.fori_loop(0, RPW // 2, pair, 0)


def _sc_pool(tabs, roist):
    mesh = plsc.VectorSubcoreMesh(core_axis_name="c", subcore_axis_name="s")
    return pl.kernel(
        _sc_body,
        mesh=mesh,
        compiler_params=pltpu.CompilerParams(needs_layout_passes=False),
        out_type=jax.ShapeDtypeStruct((NROI, C * NBIN), jnp.float32),
        scratch_types=[
            pltpu.VMEM((4, RPW), jnp.float32),
            pltpu.VMEM((RPW, NSEG, GROWS), jnp.int32),
            pltpu.VMEM((2, RROWS, C), jnp.float32),
            pltpu.VMEM((CHALF * NBIN,), jnp.float32),
            pltpu.SemaphoreType.DMA,
            pltpu.SemaphoreType.DMA,
        ],
    )(tabs, roist)


@jax.jit
def kernel(FM, rois):
    FMt = jnp.transpose(FM, (1, 2, 0))
    tabs = _build_tables(FMt).reshape(NTAB * HW, C)
    roist = jnp.transpose(rois, (1, 0))
    out = _sc_pool(tabs, roist)
    return out.reshape(NROI, C, P, P)


# speedup vs baseline: 2.3684x; 2.3589x over previous
import functools

import jax
import jax.numpy as jnp
from jax import lax
from jax.experimental import pallas as pl
from jax.experimental.pallas import tpu as pltpu
from jax.experimental.pallas import tpu_sc as plsc

H = 56
W = 56
C = 256
NROI = 512
P = 7
NBIN = P * P
NSPAN = 9
NTAB = NSPAN * 4
HW = H * W

NC = 2
NS = 16
NWORK = NC * NS
RPW = NROI // NWORK
GROWS = 104


def _tables_body(fmt_ref, out_ref):
    s = pl.program_id(0)
    F = fmt_ref[...]
    X = F
    for j in range(1, NSPAN):
        sh = jnp.concatenate(
            [F[j:], jnp.broadcast_to(F[-1:], (j, W, C))], axis=0)
        X = jnp.where(s >= j, jnp.maximum(X, sh), X)
    Y = X
    for kw in range(4):
        if kw > 0:
            d = 1 << (kw - 1)
            sh = jnp.concatenate(
                [Y[:, d:], jnp.broadcast_to(Y[:, -1:], (H, d, C))], axis=1)
            Y = jnp.maximum(Y, sh)
        out_ref[0, kw] = Y


def _build_tables(FMt):
    return pl.pallas_call(
        _tables_body,
        grid=(NSPAN,),
        in_specs=[pl.BlockSpec((H, W, C), lambda g: (0, 0, 0))],
        out_specs=pl.BlockSpec((1, 4, H, W, C), lambda g: (g, 0, 0, 0, 0)),
        out_shape=jax.ShapeDtypeStruct((NSPAN, 4, H, W, C), jnp.float32),
    )(FMt)


def _rint_nonneg(x):
    fl = x.astype(jnp.int32)
    fr = x - fl.astype(jnp.float32)
    odd = (fl & 1) == 1
    up = (fr > 0.5) | ((fr == 0.5) & odd)
    return fl + up.astype(jnp.int32)


def _sc_body(tabs, roist, out, rv, idxb, rows, outT, semA, semB):
    cid = lax.axis_index("c")
    sid = lax.axis_index("s")
    wid = sid * NC + cid
    base = wid * RPW

    for d in range(4):
        pltpu.sync_copy(roist.at[d, pl.ds(base * 1, RPW)], rv.at[d])

    lane = jnp.arange(RPW, dtype=jnp.int32)
    zero = jnp.zeros((RPW,), jnp.int32)

    def idx_store(slot, val):
        plsc.store_scatter(
            idxb,
            [lane, zero, jnp.full((RPW,), slot, jnp.int32)],
            val)

    for k in range(2 * NBIN, GROWS):
        idx_store(k, zero)

    fi = rv[0]
    fj = rv[1]
    fh = rv[2]
    fw = rv[3]
    y0 = jnp.clip(_rint_nonneg(fi * float(H)), 0, H - 1)
    x0 = jnp.clip(_rint_nonneg(fj * float(W)), 0, W - 1)
    rh = jnp.minimum(jnp.maximum(_rint_nonneg(fh * float(H)), 1), H - y0)
    rw = jnp.minimum(jnp.maximum(_rint_nonneg(fw * float(W)), 1), W - x0)

    def edges(p, v0, rv_):
        st = v0 + (p * rv_) // P
        e = v0 + ((p + 1) * rv_ + (P - 1)) // P
        e = jnp.maximum(e, st + 1)
        return st, e - st

    hA = [edges(p, y0, rh) for p in range(P)]
    wA = []
    for p in range(P):
        ws, dw = edges(p, x0, rw)
        pw = jnp.where(dw >= 8, 8,
                       jnp.where(dw >= 4, 4, jnp.where(dw >= 2, 2, 1)))
        kw = ((dw >= 2).astype(jnp.int32) + (dw >= 4).astype(jnp.int32)
              + (dw >= 8).astype(jnp.int32))
        wA.append((ws, we := ws, kw, (ws + dw) - pw))

    for py in range(P):
        hs, dh = hA[py]
        rbase = (dh - 1) * (4 * HW) + hs * W
        for px in range(P):
            ws, _, kw, c1 = wA[px]
            tb = rbase + kw * HW
            b = py * P + px
            idx_store(b, tb + ws)
            idx_store(NBIN + b, tb + c1)

    sems = (semA, semB)
    ivec = jnp.arange(16, dtype=jnp.int32) * NBIN

    def issue(r, buf):
        pltpu.async_copy(
            tabs.at[idxb.at[r, 0]],
            rows.at[buf], sems[buf])

    def drain(buf):
        pltpu.make_async_copy(
            tabs.at[pl.ds(0, GROWS)], rows.at[buf], sems[buf]).wait()

    def compute(r, buf):
        def per_row(py, carry):
            b0 = py * P
            for px in range(P):
                bvec = ivec + (b0 + px)
                for v in range(C // 16):
                    m = jnp.maximum(
                        rows[buf, b0 + px, pl.ds(16 * v, 16)],
                        rows[buf, NBIN + b0 + px, pl.ds(16 * v, 16)])
                    plsc.store_scatter(outT, [bvec + (16 * NBIN * v)], m)
            return carry

        lax.fori_loop(0, P, per_row, 0)
        pltpu.sync_copy(outT, out.at[base + r])

    issue(0, 0)
    issue(1, 1)

    def pair(g, carry):
        ra = 2 * g
        drain(0)
        compute(ra, 0)

        @pl.when(ra + 2 < RPW)
        def _():
            issue(ra + 2, 0)

        drain(1)
        compute(ra + 1, 1)

        @pl.when(ra + 3 < RPW)
        def _():
            issue(ra + 3, 1)

        return carry

    lax.fori_loop(0, RPW // 2, pair, 0)


def _sc_pool(tabs, roist):
    mesh = plsc.VectorSubcoreMesh(core_axis_name="c", subcore_axis_name="s")
    return pl.kernel(
        _sc_body,
        mesh=mesh,
        compiler_params=pltpu.CompilerParams(needs_layout_passes=False),
        out_type=jax.ShapeDtypeStruct((NROI, C * NBIN), jnp.float32),
        scratch_types=[
            pltpu.VMEM((4, RPW), jnp.float32),
            pltpu.VMEM((RPW, 1, GROWS), jnp.int32),
            pltpu.VMEM((2, GROWS, C), jnp.float32),
            pltpu.VMEM((C * NBIN,), jnp.float32),
            pltpu.SemaphoreType.DMA,
            pltpu.SemaphoreType.DMA,
        ],
    )(tabs, roist)


@jax.jit
def kernel(FM, rois):
    FMt = jnp.transpose(FM, (1, 2, 0))
    tabs = _build_tables(FMt).reshape(NTAB * HW, C)
    roist = jnp.transpose(rois, (1, 0))
    out = _sc_pool(tabs, roist)
    return out.reshape(NROI, C, P, P)
